# Initial kernel scaffold; baseline (speedup 1.0000x reference)
import jax
import jax.numpy as jnp
from jax.experimental import pallas as pl


def kernel(x, edge_index, edge_attr, batch, params):
    raise NotImplementedError("write your pallas kernel here")


# speedup vs baseline: 1.1316x; 1.1316x over previous
import functools

import jax
import jax.numpy as jnp
from jax import lax
from jax.experimental import pallas as pl
from jax.experimental.pallas import tpu as pltpu
from jax.experimental.pallas import tpu_sc as plsc

_NC = 2
_NS = 16
_NW = _NC * _NS
_CH = 128


def _sc_mesh():
    return plsc.VectorSubcoreMesh(core_axis_name="c", subcore_axis_name="s")


def _sc_gather(table, idx3, d):
    nch = idx3.shape[1]
    rows_total = _NW * nch * _CH

    @functools.partial(
        pl.kernel,
        out_type=jax.ShapeDtypeStruct((rows_total, d), jnp.float32),
        mesh=_sc_mesh(),
        scratch_types=[
            pltpu.VMEM((nch, _CH), jnp.int32),
            pltpu.VMEM((_CH, d), jnp.float32),
            pltpu.SemaphoreType.DMA,
        ],
    )
    def k(table_hbm, idx_hbm, out_hbm, idx_v, rows_v, sem):
        w = lax.axis_index("s") * _NC + lax.axis_index("c")
        pltpu.sync_copy(idx_hbm.at[w], idx_v)

        def body(j, carry):
            pltpu.async_copy(table_hbm.at[idx_v.at[j]], rows_v, sem).wait()
            pltpu.sync_copy(rows_v, out_hbm.at[pl.ds((w * nch + j) * _CH, _CH)])
            return carry

        lax.fori_loop(0, nch, body, 0)

    return k(table, idx3)


def _sc_scatter(m, idx3, zs, n_rows):
    nch = idx3.shape[1]
    tw = 9
    ts = -(-(tw * n_rows) // 128) * 128

    @functools.partial(
        pl.kernel,
        out_type=jax.ShapeDtypeStruct((_NW * ts,), jnp.float32),
        mesh=_sc_mesh(),
        compiler_params=pltpu.CompilerParams(needs_layout_passes=False),
        scratch_types=[
            pltpu.VMEM((nch, _CH), jnp.int32),
            pltpu.VMEM((_CH, 16), jnp.float32),
            pltpu.VMEM((ts,), jnp.float32),
            pltpu.SemaphoreType.DMA,
        ],
    )
    def k(m_hbm, idx_hbm, zs_hbm, out_hbm, idx_v, row_v, tab_v, sem):
        c = lax.axis_index("c")
        s = lax.axis_index("s")
        w = s * _NC + c
        lanes = lax.iota(jnp.int32, 16)
        colmask = lanes < tw

        pltpu.sync_copy(zs_hbm, tab_v)
        pltpu.sync_copy(idx_hbm.at[w], idx_v)

        def body(j, carry):
            pltpu.sync_copy(m_hbm.at[pl.ds((w * nch + j) * _CH, _CH)], row_v)
            jv = jnp.full((16,), j, jnp.int32)
            for t in range(_CH):
                vals = row_v[t, :]
                dstv = plsc.load_gather(idx_v,
                                        [jv, jnp.full((16,), t, jnp.int32)])
                tidx = lanes * n_rows + dstv
                cur = plsc.load_gather(tab_v, [tidx], mask=colmask)
                plsc.store_scatter(tab_v, [tidx], cur + vals, mask=colmask)
            return carry

        lax.fori_loop(0, nch, body, 0)
        pltpu.sync_copy(tab_v, out_hbm.at[pl.ds(w * ts, ts)])

    return k(m, idx3, zs)


def _tc_msg(ea_pad, g_all, w1, b1, w2m, b2r, n_real, be=2048):
    epad, dg = g_all.shape
    k2 = w2m.shape[0]
    hid = w1.shape[1]
    out_c = b2r.shape[1]
    grid = epad // be

    def body(ea_ref, g_ref, w1_ref, b1_ref, w2m_ref, b2r_ref, o_ref):
        hmid = jnp.maximum(
            jnp.dot(ea_ref[...], w1_ref[...], preferred_element_type=jnp.float32)
            + b1_ref[...], 0.0)
        g = g_ref[...][:, :k2]
        u = jnp.dot(g, w2m_ref[...], preferred_element_type=jnp.float32)
        bt = jnp.dot(g, b2r_ref[...], preferred_element_type=jnp.float32)
        parts = [
            jnp.sum(hmid * u[:, o * hid:(o + 1) * hid], axis=1, keepdims=True)
            for o in range(out_c)
        ]
        msg = jnp.concatenate(parts, axis=1) + bt
        i = pl.program_id(0)
        rows = i * be + lax.broadcasted_iota(jnp.int32, (be, 1), 0)
        valid = (rows < n_real).astype(jnp.float32)
        o_ref[...] = jnp.concatenate(
            [msg * valid, valid, jnp.zeros((be, 15 - out_c), jnp.float32)], axis=1)

    return pl.pallas_call(
        body,
        grid=(grid,),
        in_specs=[
            pl.BlockSpec((be, ea_pad.shape[1]), lambda i: (i, 0)),
            pl.BlockSpec((be, dg), lambda i: (i, 0)),
            pl.BlockSpec(w1.shape, lambda i: (0, 0)),
            pl.BlockSpec(b1.shape, lambda i: (0, 0)),
            pl.BlockSpec(w2m.shape, lambda i: (0, 0)),
            pl.BlockSpec(b2r.shape, lambda i: (0, 0)),
        ],
        out_specs=pl.BlockSpec((be, 16), lambda i: (i, 0)),
        out_shape=jax.ShapeDtypeStruct((epad, 16), jnp.float32),
    )(ea_pad, g_all, w1, b1, w2m, b2r)


def _tc_combine(p_all, h_in, root, bias, bn_g, bn_b, out_w):
    n = h_in.shape[0]

    def body(p_ref, x_ref, r_ref, b_ref, g_ref, bb_ref, o_ref):
        psum = jnp.sum(p_ref[...], axis=0)
        ssum = psum[:8, :n]
        cnt = psum[8:9, :n]
        agg = ssum / jnp.maximum(cnt, 1.0)
        root_t = lax.dot_general(r_ref[...], x_ref[...],
                                 (((0,), (1,)), ((), ())),
                                 preferred_element_type=jnp.float32)
        h = agg + root_t + b_ref[...]
        mu = jnp.mean(h, axis=1, keepdims=True)
        var = jnp.mean((h - mu) ** 2, axis=1, keepdims=True)
        h = g_ref[...] * (h - mu) / jnp.sqrt(var + 1e-5) + bb_ref[...]
        h = jnp.maximum(h, 0.0)
        eye = (lax.broadcasted_iota(jnp.int32, (8, out_w), 0)
               == lax.broadcasted_iota(jnp.int32, (8, out_w), 1)
               ).astype(jnp.float32)
        o_ref[...] = lax.dot_general(h, eye, (((0,), (0,)), ((), ())),
                                     preferred_element_type=jnp.float32)

    return pl.pallas_call(
        body,
        out_shape=jax.ShapeDtypeStruct((n, out_w), jnp.float32),
    )(p_all, h_in, root, bias, bn_g, bn_b)


def _tc_final(p_all, h0p, root, bias, bn_g, bn_b, batch2d, mw1, mb1, mw2, mb2,
              n_graphs):
    n = h0p.shape[0]

    def body(p_ref, h0_ref, r_ref, b_ref, g_ref, bb_ref, bt_ref,
             w1_ref, b1_ref, w2_ref, b2_ref, o_ref):
        psum = jnp.sum(p_ref[...], axis=0)
        ssum = psum[:8, :n]
        cnt = psum[8:9, :n]
        agg = ssum / jnp.maximum(cnt, 1.0)
        h0 = h0_ref[...][:, :8]
        root_t = lax.dot_general(r_ref[...], h0, (((0,), (1,)), ((), ())),
                                 preferred_element_type=jnp.float32)
        h = agg + root_t + b_ref[...]
        mu = jnp.mean(h, axis=1, keepdims=True)
        var = jnp.mean((h - mu) ** 2, axis=1, keepdims=True)
        h = g_ref[...] * (h - mu) / jnp.sqrt(var + 1e-5) + bb_ref[...]
        h = jnp.maximum(h, 0.0)
        gid = lax.broadcasted_iota(jnp.int32, (n_graphs, n), 0)
        oh = (gid == bt_ref[...]).astype(jnp.float32)
        sums_t = lax.dot_general(h, oh, (((1,), (1,)), ((), ())),
                                 preferred_element_type=jnp.float32)
        ones = jnp.ones((1, n), jnp.float32)
        cnts_t = lax.dot_general(ones, oh, (((1,), (1,)), ((), ())),
                                 preferred_element_type=jnp.float32)
        pooled_t = sums_t / jnp.maximum(cnts_t, 1.0)
        hid_t = jnp.maximum(
            lax.dot_general(w1_ref[...], pooled_t, (((0,), (0,)), ((), ())),
                            preferred_element_type=jnp.float32)
            + b1_ref[...], 0.0)
        out_t = lax.dot_general(w2_ref[...], hid_t, (((0,), (0,)), ((), ())),
                                preferred_element_type=jnp.float32) \
            + b2_ref[...]
        eye_g = (lax.broadcasted_iota(jnp.int32, (n_graphs, n_graphs), 0)
                 == lax.broadcasted_iota(jnp.int32, (n_graphs, n_graphs), 1)
                 ).astype(jnp.float32)
        o_ref[...] = lax.dot_general(eye_g, out_t, (((1,), (1,)), ((), ())),
                                     preferred_element_type=jnp.float32)

    return pl.pallas_call(
        body,
        out_shape=jax.ShapeDtypeStruct((n_graphs, mw2.shape[1]), jnp.float32),
    )(p_all, h0p, root, bias, bn_g, bn_b, batch2d, mw1, mb1, mw2, mb2)


def kernel(x, edge_index, edge_attr, batch, params):
    n, d_in = x.shape
    e = edge_index.shape[1]
    nch = -(-e // (_NW * _CH))
    epad = _NW * nch * _CH
    pad = epad - e

    src = jnp.concatenate([edge_index[0], jnp.zeros((pad,), jnp.int32)])
    dst = jnp.concatenate([edge_index[1], jnp.zeros((pad,), jnp.int32)])
    src3 = src.reshape(_NW, nch, _CH)
    dst3 = dst.reshape(_NW, nch, _CH)
    ea_pad = jnp.concatenate(
        [edge_attr, jnp.zeros((pad, edge_attr.shape[1]), jnp.float32)], axis=0)

    p = params
    hid = p['l0_e_w1'].shape[1]
    out_c = p['l0_root'].shape[1]
    w2m0 = p['l0_e_w2'].reshape(hid, d_in, out_c).transpose(1, 2, 0) \
        .reshape(d_in, out_c * hid)
    b2r0 = p['l0_e_b2'].reshape(d_in, out_c)
    w2m1 = p['l1_e_w2'].reshape(hid, out_c, out_c).transpose(1, 2, 0) \
        .reshape(out_c, out_c * hid)
    b2r1 = p['l1_e_b2'].reshape(out_c, out_c)
    row = lambda v: v.reshape(1, -1)
    col = lambda v: v.reshape(-1, 1)

    npad = -(-n // 16) * 16
    ts = -(-(9 * npad) // 128) * 128
    zs = jnp.zeros((ts,), jnp.float32)
    unpack = lambda v: v.reshape(_NW, ts)[:, :9 * npad].reshape(_NW, 9, npad)

    g0 = _sc_gather(x, src3, d_in)
    m0 = _tc_msg(ea_pad, g0, p['l0_e_w1'], row(p['l0_e_b1']), w2m0, b2r0, e)
    p0 = unpack(_sc_scatter(m0, dst3, zs, npad))
    h0p = _tc_combine(p0, x, p['l0_root'], col(p['l0_bias']),
                      col(p['l0_bn_g']), col(p['l0_bn_b']), 128)
    g1 = _sc_gather(h0p, src3, 128)
    m1 = _tc_msg(ea_pad, g1, p['l1_e_w1'], row(p['l1_e_b1']), w2m1, b2r1, e)
    p1 = unpack(_sc_scatter(m1, dst3, zs, npad))
    return _tc_final(p1, h0p, p['l1_root'], col(p['l1_bias']),
                     col(p['l1_bn_g']), col(p['l1_bn_b']),
                     batch.reshape(1, -1), p['mlp_w1'], col(p['mlp_b1']),
                     p['mlp_w2'], col(p['mlp_b2']), 64)


# speedup vs baseline: 1.1521x; 1.0181x over previous
import functools

import jax
import jax.numpy as jnp
from jax import lax
from jax.experimental import pallas as pl
from jax.experimental.pallas import tpu as pltpu
from jax.experimental.pallas import tpu_sc as plsc

_NC = 2
_NS = 16
_NW = _NC * _NS
_CH = 128


def _sc_mesh():
    return plsc.VectorSubcoreMesh(core_axis_name="c", subcore_axis_name="s")


def _sc_gather(table, idx3, d):
    nch = idx3.shape[1]
    rows_total = _NW * nch * _CH

    @functools.partial(
        pl.kernel,
        out_type=jax.ShapeDtypeStruct((rows_total, d), jnp.float32),
        mesh=_sc_mesh(),
        scratch_types=[
            pltpu.VMEM((nch, _CH), jnp.int32),
            pltpu.VMEM((_CH, d), jnp.float32),
            pltpu.SemaphoreType.DMA,
        ],
    )
    def k(table_hbm, idx_hbm, out_hbm, idx_v, rows_v, sem):
        w = lax.axis_index("s") * _NC + lax.axis_index("c")
        pltpu.sync_copy(idx_hbm.at[w], idx_v)

        def body(j, carry):
            pltpu.async_copy(table_hbm.at[idx_v.at[j]], rows_v, sem).wait()
            pltpu.sync_copy(rows_v, out_hbm.at[pl.ds((w * nch + j) * _CH, _CH)])
            return carry

        lax.fori_loop(0, nch, body, 0)

    return k(table, idx3)


def _sc_scatter(m, idx3, zs, n_rows):
    nch = idx3.shape[1]
    tw = 9
    ts = -(-(tw * n_rows) // 128) * 128

    @functools.partial(
        pl.kernel,
        out_type=jax.ShapeDtypeStruct((_NW * ts,), jnp.float32),
        mesh=_sc_mesh(),
        compiler_params=pltpu.CompilerParams(needs_layout_passes=False),
        scratch_types=[
            pltpu.VMEM((nch, _CH), jnp.int32),
            pltpu.VMEM((_CH, 16), jnp.float32),
            pltpu.VMEM((ts,), jnp.float32),
            pltpu.SemaphoreType.DMA,
        ],
    )
    def k(m_hbm, idx_hbm, zs_hbm, out_hbm, idx_v, row_v, tab_v, sem):
        c = lax.axis_index("c")
        s = lax.axis_index("s")
        w = s * _NC + c
        lanes = lax.iota(jnp.int32, 16)
        colmask = lanes < tw

        pltpu.sync_copy(zs_hbm, tab_v)
        pltpu.sync_copy(idx_hbm.at[w], idx_v)

        def body(j, carry):
            pltpu.sync_copy(m_hbm.at[pl.ds((w * nch + j) * _CH, _CH)], row_v)
            jv = jnp.full((16,), j, jnp.int32)
            for t in range(_CH):
                vals = row_v[t, :]
                dstv = plsc.load_gather(idx_v,
                                        [jv, jnp.full((16,), t, jnp.int32)])
                tidx = lanes * n_rows + dstv
                plsc.addupdate_scatter(tab_v, [tidx], vals, mask=colmask)
            return carry

        lax.fori_loop(0, nch, body, 0)
        pltpu.sync_copy(tab_v, out_hbm.at[pl.ds(w * ts, ts)])

    return k(m, idx3, zs)


def _tc_msg(ea_pad, g_all, w1, b1, w2m, b2r, n_real, be=2048):
    epad, dg = g_all.shape
    k2 = w2m.shape[0]
    hid = w1.shape[1]
    out_c = b2r.shape[1]
    grid = epad // be

    def body(ea_ref, g_ref, w1_ref, b1_ref, w2m_ref, b2r_ref, o_ref):
        hmid = jnp.maximum(
            jnp.dot(ea_ref[...], w1_ref[...], preferred_element_type=jnp.float32)
            + b1_ref[...], 0.0)
        g = g_ref[...][:, :k2]
        u = jnp.dot(g, w2m_ref[...], preferred_element_type=jnp.float32)
        bt = jnp.dot(g, b2r_ref[...], preferred_element_type=jnp.float32)
        parts = [
            jnp.sum(hmid * u[:, o * hid:(o + 1) * hid], axis=1, keepdims=True)
            for o in range(out_c)
        ]
        msg = jnp.concatenate(parts, axis=1) + bt
        i = pl.program_id(0)
        rows = i * be + lax.broadcasted_iota(jnp.int32, (be, 1), 0)
        valid = (rows < n_real).astype(jnp.float32)
        o_ref[...] = jnp.concatenate(
            [msg * valid, valid, jnp.zeros((be, 15 - out_c), jnp.float32)], axis=1)

    return pl.pallas_call(
        body,
        grid=(grid,),
        in_specs=[
            pl.BlockSpec((be, ea_pad.shape[1]), lambda i: (i, 0)),
            pl.BlockSpec((be, dg), lambda i: (i, 0)),
            pl.BlockSpec(w1.shape, lambda i: (0, 0)),
            pl.BlockSpec(b1.shape, lambda i: (0, 0)),
            pl.BlockSpec(w2m.shape, lambda i: (0, 0)),
            pl.BlockSpec(b2r.shape, lambda i: (0, 0)),
        ],
        out_specs=pl.BlockSpec((be, 16), lambda i: (i, 0)),
        out_shape=jax.ShapeDtypeStruct((epad, 16), jnp.float32),
    )(ea_pad, g_all, w1, b1, w2m, b2r)


def _tc_combine(p_all, h_in, root, bias, bn_g, bn_b, out_w):
    n = h_in.shape[0]

    def body(p_ref, x_ref, r_ref, b_ref, g_ref, bb_ref, o_ref):
        psum = jnp.sum(p_ref[...], axis=0)
        ssum = psum[:8, :n]
        cnt = psum[8:9, :n]
        agg = ssum / jnp.maximum(cnt, 1.0)
        root_t = lax.dot_general(r_ref[...], x_ref[...],
                                 (((0,), (1,)), ((), ())),
                                 preferred_element_type=jnp.float32)
        h = agg + root_t + b_ref[...]
        mu = jnp.mean(h, axis=1, keepdims=True)
        var = jnp.mean((h - mu) ** 2, axis=1, keepdims=True)
        h = g_ref[...] * (h - mu) / jnp.sqrt(var + 1e-5) + bb_ref[...]
        h = jnp.maximum(h, 0.0)
        eye = (lax.broadcasted_iota(jnp.int32, (8, out_w), 0)
               == lax.broadcasted_iota(jnp.int32, (8, out_w), 1)
               ).astype(jnp.float32)
        o_ref[...] = lax.dot_general(h, eye, (((0,), (0,)), ((), ())),
                                     preferred_element_type=jnp.float32)

    return pl.pallas_call(
        body,
        out_shape=jax.ShapeDtypeStruct((n, out_w), jnp.float32),
    )(p_all, h_in, root, bias, bn_g, bn_b)


def _tc_final(p_all, h0p, root, bias, bn_g, bn_b, batch2d, mw1, mb1, mw2, mb2,
              n_graphs):
    n = h0p.shape[0]

    def body(p_ref, h0_ref, r_ref, b_ref, g_ref, bb_ref, bt_ref,
             w1_ref, b1_ref, w2_ref, b2_ref, o_ref):
        psum = jnp.sum(p_ref[...], axis=0)
        ssum = psum[:8, :n]
        cnt = psum[8:9, :n]
        agg = ssum / jnp.maximum(cnt, 1.0)
        h0 = h0_ref[...][:, :8]
        root_t = lax.dot_general(r_ref[...], h0, (((0,), (1,)), ((), ())),
                                 preferred_element_type=jnp.float32)
        h = agg + root_t + b_ref[...]
        mu = jnp.mean(h, axis=1, keepdims=True)
        var = jnp.mean((h - mu) ** 2, axis=1, keepdims=True)
        h = g_ref[...] * (h - mu) / jnp.sqrt(var + 1e-5) + bb_ref[...]
        h = jnp.maximum(h, 0.0)
        gid = lax.broadcasted_iota(jnp.int32, (n_graphs, n), 0)
        oh = (gid == bt_ref[...]).astype(jnp.float32)
        sums_t = lax.dot_general(h, oh, (((1,), (1,)), ((), ())),
                                 preferred_element_type=jnp.float32)
        ones = jnp.ones((1, n), jnp.float32)
        cnts_t = lax.dot_general(ones, oh, (((1,), (1,)), ((), ())),
                                 preferred_element_type=jnp.float32)
        pooled_t = sums_t / jnp.maximum(cnts_t, 1.0)
        hid_t = jnp.maximum(
            lax.dot_general(w1_ref[...], pooled_t, (((0,), (0,)), ((), ())),
                            preferred_element_type=jnp.float32)
            + b1_ref[...], 0.0)
        out_t = lax.dot_general(w2_ref[...], hid_t, (((0,), (0,)), ((), ())),
                                preferred_element_type=jnp.float32) \
            + b2_ref[...]
        eye_g = (lax.broadcasted_iota(jnp.int32, (n_graphs, n_graphs), 0)
                 == lax.broadcasted_iota(jnp.int32, (n_graphs, n_graphs), 1)
                 ).astype(jnp.float32)
        o_ref[...] = lax.dot_general(eye_g, out_t, (((1,), (1,)), ((), ())),
                                     preferred_element_type=jnp.float32)

    return pl.pallas_call(
        body,
        out_shape=jax.ShapeDtypeStruct((n_graphs, mw2.shape[1]), jnp.float32),
    )(p_all, h0p, root, bias, bn_g, bn_b, batch2d, mw1, mb1, mw2, mb2)


def kernel(x, edge_index, edge_attr, batch, params):
    n, d_in = x.shape
    e = edge_index.shape[1]
    nch = -(-e // (_NW * _CH))
    epad = _NW * nch * _CH
    pad = epad - e

    src = jnp.concatenate([edge_index[0], jnp.zeros((pad,), jnp.int32)])
    dst = jnp.concatenate([edge_index[1], jnp.zeros((pad,), jnp.int32)])
    src3 = src.reshape(_NW, nch, _CH)
    dst3 = dst.reshape(_NW, nch, _CH)
    ea_pad = jnp.concatenate(
        [edge_attr, jnp.zeros((pad, edge_attr.shape[1]), jnp.float32)], axis=0)

    p = params
    hid = p['l0_e_w1'].shape[1]
    out_c = p['l0_root'].shape[1]
    w2m0 = p['l0_e_w2'].reshape(hid, d_in, out_c).transpose(1, 2, 0) \
        .reshape(d_in, out_c * hid)
    b2r0 = p['l0_e_b2'].reshape(d_in, out_c)
    w2m1 = p['l1_e_w2'].reshape(hid, out_c, out_c).transpose(1, 2, 0) \
        .reshape(out_c, out_c * hid)
    b2r1 = p['l1_e_b2'].reshape(out_c, out_c)
    row = lambda v: v.reshape(1, -1)
    col = lambda v: v.reshape(-1, 1)

    npad = -(-n // 16) * 16
    ts = -(-(9 * npad) // 128) * 128
    zs = jnp.zeros((ts,), jnp.float32)
    unpack = lambda v: v.reshape(_NW, ts)[:, :9 * npad].reshape(_NW, 9, npad)

    g0 = _sc_gather(x, src3, d_in)
    m0 = _tc_msg(ea_pad, g0, p['l0_e_w1'], row(p['l0_e_b1']), w2m0, b2r0, e)
    p0 = unpack(_sc_scatter(m0, dst3, zs, npad))
    h0p = _tc_combine(p0, x, p['l0_root'], col(p['l0_bias']),
                      col(p['l0_bn_g']), col(p['l0_bn_b']), 128)
    g1 = _sc_gather(h0p, src3, 128)
    m1 = _tc_msg(ea_pad, g1, p['l1_e_w1'], row(p['l1_e_b1']), w2m1, b2r1, e)
    p1 = unpack(_sc_scatter(m1, dst3, zs, npad))
    return _tc_final(p1, h0p, p['l1_root'], col(p['l1_bias']),
                     col(p['l1_bn_g']), col(p['l1_bn_b']),
                     batch.reshape(1, -1), p['mlp_w1'], col(p['mlp_b1']),
                     p['mlp_w2'], col(p['mlp_b2']), 64)
